# baseline (speedup 1.0000x reference)
import functools

import jax
import jax.numpy as jnp
from jax import lax
from jax.experimental import pallas as pl
from jax.experimental.pallas import tpu as pltpu
from jax.experimental.pallas import tpu_sc as plsc

_N_WORDS = 100000
_DIM = 16
_CPAD = 96
_NT = 32

_COV_ROWS = _N_WORDS
_SUM_ROWS = _N_WORDS // 8
_CNT_ROWS = (_N_WORDS + _CPAD) // 128

_COV_CHUNK = (_COV_ROWS // _NT) & ~7
_COV_TAIL = _COV_ROWS - _COV_CHUNK * _NT
_SUM_CHUNK = (_SUM_ROWS // _NT) & ~7
_SUM_TAIL = _SUM_ROWS - _SUM_CHUNK * _NT
_CNT_CHUNK = _CNT_ROWS // _NT
_CNT_TAIL = _CNT_ROWS - _CNT_CHUNK * _NT

_mesh = plsc.VectorSubcoreMesh(core_axis_name="c", subcore_axis_name="s")


@functools.partial(
    pl.kernel,
    out_type=(
        jax.ShapeDtypeStruct((_SUM_ROWS, 128), jnp.float32),
        jax.ShapeDtypeStruct((_CNT_ROWS, 128), jnp.int32),
        jax.ShapeDtypeStruct((_COV_ROWS, 256), jnp.float32),
    ),
    mesh=_mesh,
    scratch_types=[
        pltpu.VMEM((1,), jnp.int32),
        pltpu.VMEM((1,), jnp.int32),
        pltpu.VMEM((1,), jnp.int32),
        pltpu.VMEM((6, _DIM), jnp.int32),
        pltpu.VMEM((_DIM,), jnp.float32),
        pltpu.VMEM((1, 128), jnp.float32),
        pltpu.VMEM((1, 128), jnp.int32),
        pltpu.VMEM((1, 16 * _DIM), jnp.float32),
        pltpu.SemaphoreType.DMA,
    ],
)
def _sc_all(idxs_hbm, idxc_hbm, idxv_hbm, bc_hbm, vec_hbm,
            sum_in, cnt_in, cov_in,
            sum_out, cnt_out, cov_out,
            idxs_v, idxc_v, idxv_v, bc_v, vec_v, sum_v, cnt_v, cov_v, sem):
    cid = lax.axis_index("c")
    sid = lax.axis_index("s")
    wid = sid * 2 + cid

    pltpu.sync_copy(idxs_hbm, idxs_v)
    pltpu.sync_copy(idxc_hbm, idxc_v)
    pltpu.sync_copy(idxv_hbm, idxv_v)
    pltpu.sync_copy(bc_hbm, bc_v)
    pltpu.sync_copy(vec_hbm, vec_v)

    ix = bc_v[3, :][0]
    rs = bc_v[4, :][0]
    rc = bc_v[5, :][0]

    own_cov = jnp.where(ix >= _COV_CHUNK * _NT, 0, ix // _COV_CHUNK)
    own_sum = jnp.where(rs >= _SUM_CHUNK * _NT, 0, rs // _SUM_CHUNK)
    own_cnt = jnp.where(rc >= _CNT_CHUNK * _NT, 0, rc // _CNT_CHUNK)

    c_cov = pltpu.async_copy(
        cov_in.at[pl.ds(wid * _COV_CHUNK, _COV_CHUNK)],
        cov_out.at[pl.ds(wid * _COV_CHUNK, _COV_CHUNK)], sem)
    c_sum = pltpu.async_copy(
        sum_in.at[pl.ds(wid * _SUM_CHUNK, _SUM_CHUNK)],
        sum_out.at[pl.ds(wid * _SUM_CHUNK, _SUM_CHUNK)], sem)
    c_cnt = pltpu.async_copy(
        cnt_in.at[pl.ds(wid * _CNT_CHUNK, _CNT_CHUNK)],
        cnt_out.at[pl.ds(wid * _CNT_CHUNK, _CNT_CHUNK)], sem)
    c_cov.wait()
    c_sum.wait()
    c_cnt.wait()

    @pl.when(wid == 0)
    def _():
        t_cov = pltpu.async_copy(
            cov_in.at[pl.ds(_COV_CHUNK * _NT, _COV_TAIL)],
            cov_out.at[pl.ds(_COV_CHUNK * _NT, _COV_TAIL)], sem)
        t_sum = pltpu.async_copy(
            sum_in.at[pl.ds(_SUM_CHUNK * _NT, _SUM_TAIL)],
            sum_out.at[pl.ds(_SUM_CHUNK * _NT, _SUM_TAIL)], sem)
        t_cnt = pltpu.async_copy(
            cnt_in.at[pl.ds(_CNT_CHUNK * _NT, _CNT_TAIL)],
            cnt_out.at[pl.ds(_CNT_CHUNK * _NT, _CNT_TAIL)], sem)
        t_cov.wait()
        t_sum.wait()
        t_cnt.wait()

    vec = vec_v[...]
    rem8 = bc_v[0, :]
    grp = bc_v[1, :]
    lane = bc_v[2, :]
    iota = lax.iota(jnp.int32, _DIM)
    zf = jnp.zeros((_DIM,), jnp.float32)

    @pl.when(wid == own_sum)
    def _():
        pltpu.async_copy(sum_out.at[idxs_v], sum_v, sem).wait()
        for j in range(8):
            s = pl.ds(j * _DIM, _DIM)
            sum_v[0, s] = sum_v[0, s] + jnp.where(rem8 == j, vec, zf)
        pltpu.async_copy(sum_v, sum_out.at[idxs_v], sem).wait()

    @pl.when(wid == own_cnt)
    def _():
        pltpu.async_copy(cnt_out.at[idxc_v], cnt_v, sem).wait()
        for j in range(8):
            s = pl.ds(j * _DIM, _DIM)
            hit = jnp.logical_and(grp == j, iota == lane)
            cnt_v[0, s] = cnt_v[0, s] + jnp.where(hit, 1, 0)
        pltpu.async_copy(cnt_v, cnt_out.at[idxc_v], sem).wait()

    @pl.when(wid == own_cov)
    def _():
        pltpu.async_copy(cov_out.at[idxv_v], cov_v, sem).wait()
        for j in range(_DIM):
            s = pl.ds(j * _DIM, _DIM)
            cov_v[0, s] = cov_v[0, s] + vec * vec[j]
        pltpu.async_copy(cov_v, cov_out.at[idxv_v], sem).wait()


def kernel(ix, vec, sum_buf, counts, cov_buf):
    ix32 = jnp.asarray(ix, jnp.int32)
    idxs = jnp.reshape(ix32 // 8, (1,))
    idxc = jnp.reshape(ix32 // 128, (1,))
    idxv = jnp.reshape(ix32, (1,))
    bc = jnp.stack([
        jnp.full((_DIM,), ix32 % 8, jnp.int32),
        jnp.full((_DIM,), (ix32 % 128) // _DIM, jnp.int32),
        jnp.full((_DIM,), ix32 % _DIM, jnp.int32),
        jnp.full((_DIM,), ix32, jnp.int32),
        jnp.full((_DIM,), ix32 // 8, jnp.int32),
        jnp.full((_DIM,), ix32 // 128, jnp.int32),
    ])
    cpad = jnp.concatenate([counts, jnp.zeros((_CPAD,), jnp.int32)])
    new_sum, new_cnt, new_cov = _sc_all(
        idxs, idxc, idxv, bc, vec,
        sum_buf.reshape(_SUM_ROWS, 128),
        cpad.reshape(_CNT_ROWS, 128),
        cov_buf.reshape(_COV_ROWS, 256),
    )
    return (new_sum.reshape(_N_WORDS, _DIM),
            new_cnt.reshape(-1)[:_N_WORDS],
            new_cov.reshape(_N_WORDS, _DIM, _DIM))


# speedup vs baseline: 48.5390x; 48.5390x over previous
import jax
import jax.numpy as jnp
from jax import lax
from jax.experimental import pallas as pl
from jax.experimental.pallas import tpu as pltpu

_N_WORDS = 100000
_DIM = 16
_CNT_ROWS = 782
_CPAD = _CNT_ROWS * 128 - _N_WORDS


def _body(ix_ref, vecb_in, vhi_in, vlo_in, sum_in, cnt_in, cov_in,
          sum_out, cnt_out, cov_out):
    ix = ix_ref[0]
    c = ix % 128

    lane16 = lax.broadcasted_iota(jnp.int32, (_DIM, 128), 1)
    sum_out[...] = sum_in[...] + jnp.where(lane16 == c, vecb_in[...], 0.0)

    r2 = (ix // 128) % 8
    row8 = lax.broadcasted_iota(jnp.int32, (8, 128), 0)
    lane8 = lax.broadcasted_iota(jnp.int32, (8, 128), 1)
    hit_c = jnp.logical_and(row8 == r2, lane8 == c)
    cnt_out[...] = cnt_in[...] + hit_c.astype(jnp.int32)

    lane256 = lax.broadcasted_iota(jnp.int32, (16 * _DIM, 128), 1)
    outer = vhi_in[...] * vlo_in[...]
    cov_out[...] = cov_in[...] + jnp.where(lane256 == c, outer, 0.0)


def kernel(ix, vec, sum_buf, counts, cov_buf):
    ix_arr = jnp.reshape(jnp.asarray(ix, jnp.int32), (1,))
    sum_t = jnp.transpose(sum_buf, (1, 0))
    cov_t = jnp.transpose(cov_buf, (1, 2, 0)).reshape(16 * _DIM, _N_WORDS)
    cpad = jnp.concatenate(
        [counts, jnp.zeros((_CPAD,), jnp.int32)]).reshape(_CNT_ROWS, 128)
    vecb = jnp.broadcast_to(vec.reshape(_DIM, 1), (_DIM, 128))
    vhi = jnp.broadcast_to(
        vec.reshape(_DIM, 1, 1), (_DIM, _DIM, 128)).reshape(16 * _DIM, 128)
    vlo = jnp.broadcast_to(
        vec.reshape(1, _DIM, 1), (_DIM, _DIM, 128)).reshape(16 * _DIM, 128)
    grid_spec = pltpu.PrefetchScalarGridSpec(
        num_scalar_prefetch=1,
        grid=(1,),
        in_specs=[
            pl.BlockSpec((_DIM, 128), lambda i, s: (0, 0)),
            pl.BlockSpec((16 * _DIM, 128), lambda i, s: (0, 0)),
            pl.BlockSpec((16 * _DIM, 128), lambda i, s: (0, 0)),
            pl.BlockSpec((_DIM, 128), lambda i, s: (0, s[0] // 128)),
            pl.BlockSpec((8, 128), lambda i, s: (s[0] // 1024, 0)),
            pl.BlockSpec((16 * _DIM, 128), lambda i, s: (0, s[0] // 128)),
        ],
        out_specs=[
            pl.BlockSpec((_DIM, 128), lambda i, s: (0, s[0] // 128)),
            pl.BlockSpec((8, 128), lambda i, s: (s[0] // 1024, 0)),
            pl.BlockSpec((16 * _DIM, 128), lambda i, s: (0, s[0] // 128)),
        ],
    )
    out = pl.pallas_call(
        _body,
        grid_spec=grid_spec,
        out_shape=[
            jax.ShapeDtypeStruct((_DIM, _N_WORDS), jnp.float32),
            jax.ShapeDtypeStruct((_CNT_ROWS, 128), jnp.int32),
            jax.ShapeDtypeStruct((16 * _DIM, _N_WORDS), jnp.float32),
        ],
        input_output_aliases={4: 0, 5: 1, 6: 2},
    )(ix_arr, vecb, vhi, vlo, sum_t, cpad, cov_t)
    return (jnp.transpose(out[0], (1, 0)),
            out[1].reshape(-1)[:_N_WORDS],
            jnp.transpose(out[2].reshape(_DIM, _DIM, _N_WORDS), (2, 0, 1)))
